# baseline (speedup 1.0000x reference)
import functools

import jax
import jax.numpy as jnp
from jax import lax
from jax.experimental import pallas as pl
from jax.experimental.pallas import tpu as pltpu
from jax.experimental.pallas import tpu_sc as plsc

_LANES = 16
_RING = 4


@functools.lru_cache(maxsize=None)
def _build(n_tok, vocab, hidden, max_pos):
    info = plsc.get_sparse_core_info()
    num_workers = info.num_cores * info.num_subcores
    assert n_tok % num_workers == 0
    tokens_per_worker = n_tok // num_workers
    chunk = 8
    assert tokens_per_worker % (_RING * chunk) == 0
    n_chunks = tokens_per_worker // chunk
    n_vec = hidden // _LANES

    mesh = plsc.VectorSubcoreMesh(core_axis_name="c", subcore_axis_name="s")

    row_buf = pltpu.VMEM((chunk, hidden), jnp.float32)

    @functools.partial(
        pl.kernel,
        mesh=mesh,
        out_type=jax.ShapeDtypeStruct((n_tok, hidden), jnp.float32),
        scratch_types=(
            [pltpu.VMEM((tokens_per_worker,), jnp.int32)] * 2
            + [row_buf] * (2 * _RING)
            + [pltpu.SemaphoreType.DMA] * (3 * _RING)
        ),
    )
    def emb_kernel(ids_hbm, pids_hbm, word_hbm, pos_hbm, out_hbm, *refs):
        idw, idp = refs[0], refs[1]
        ob = refs[2:2 + _RING]
        pb = refs[2 + _RING:2 + 2 * _RING]
        sems = refs[2 + 2 * _RING:]
        sw = sems[0:_RING]
        sp = sems[_RING:2 * _RING]
        so = sems[2 * _RING:3 * _RING]

        wid = lax.axis_index("s") * info.num_cores + lax.axis_index("c")
        base = wid * tokens_per_worker
        pltpu.sync_copy(ids_hbm.at[pl.ds(base, tokens_per_worker)], idw)
        pltpu.sync_copy(pids_hbm.at[pl.ds(base, tokens_per_worker)], idp)

        def issue(c, b):
            off = pl.multiple_of(c * chunk, chunk)
            pltpu.async_copy(word_hbm.at[idw.at[pl.ds(off, chunk)]], ob[b], sw[b])
            pltpu.async_copy(pos_hbm.at[idp.at[pl.ds(off, chunk)]], pb[b], sp[b])

        def wait_gathers(b):
            pltpu.make_async_copy(
                word_hbm.at[idw.at[pl.ds(0, chunk)]], ob[b], sw[b]).wait()
            pltpu.make_async_copy(
                pos_hbm.at[idp.at[pl.ds(0, chunk)]], pb[b], sp[b]).wait()

        def wait_scatter(b):
            pltpu.make_async_copy(
                ob[b], out_hbm.at[pl.ds(base, chunk)], so[b]).wait()

        def add_rows(b):
            obuf, pbuf = ob[b], pb[b]

            def row(t, acc):
                for j in range(n_vec):
                    sl = pl.ds(j * _LANES, _LANES)
                    plsc.addupdate(obuf.at[t, sl], pbuf[t, sl])
                return acc

            lax.fori_loop(0, chunk, row, 0)

        def step(c, b):
            wait_gathers(b)
            add_rows(b)
            out_off = pl.multiple_of(base + c * chunk, chunk)
            pltpu.async_copy(ob[b], out_hbm.at[pl.ds(out_off, chunk)], so[b])

            b2 = (b + 2) % _RING

            @pl.when(c >= 2)
            def _():
                wait_scatter(b2)

            @pl.when(c + 2 < n_chunks)
            def _():
                issue(c + 2, b2)

        issue(0, 0)
        issue(1, 1)

        def body(cc, carry):
            c0 = cc * _RING
            for b in range(_RING):
                step(c0 + b, b)
            return carry

        lax.fori_loop(0, n_chunks // _RING, body, 0)
        wait_scatter((n_chunks - 1) % _RING)
        wait_scatter((n_chunks - 2) % _RING)

    return emb_kernel


def kernel(input_ids, position_ids, word_embeddings_weight, position_embeddings_weight):
    b, s = input_ids.shape
    vocab, hidden = word_embeddings_weight.shape
    max_pos = position_embeddings_weight.shape[0]
    fn = _build(b * s, vocab, hidden, max_pos)
    out = fn(
        input_ids.reshape(-1),
        position_ids.reshape(-1),
        word_embeddings_weight,
        position_embeddings_weight,
    )
    return out.reshape(b, s, hidden)
